# baseline (speedup 1.0000x reference)
import functools
import math

import jax
import jax.numpy as jnp
import numpy as np
from jax import lax
from jax.experimental import pallas as pl
from jax.experimental.pallas import tpu as pltpu
from jax.experimental.pallas import tpu_sc as plsc

_BATCH = 1024
_SEQ = 200
_EMB = 64
_FLAT = _BATCH * _SEQ
_NW = 32
_PER_W = _FLAT // _NW
_CHUNK = 2 * _SEQ
_NCHUNK = _PER_W // _CHUNK
_NPAIR = _NCHUNK // 2
_SCALE = math.sqrt(_EMB)


def _pe_tiled():
    position = np.arange(_SEQ, dtype=np.float32)[:, None]
    div_term = np.exp(
        np.arange(0, _EMB, 2, dtype=np.float32) * (-math.log(10000.0) / _EMB))
    pe = np.zeros((_SEQ, _EMB), dtype=np.float32)
    pe[:, 0::2] = np.sin(position * div_term)
    pe[:, 1::2] = np.cos(position * div_term)
    return np.tile(pe, (2, 1))


_PE2 = _pe_tiled()


def _compute_chunk(buf, off, mask_v, pe_v):
    dnums = lax.GatherDimensionNumbers(
        offset_dims=(), collapsed_slice_dims=(0,), start_index_map=(0,))

    def row_block(i, carry):
        r0 = i * 16
        m16 = mask_v[pl.ds(off + r0, 16)]
        for u in range(16):
            r = r0 + u
            lane = jnp.full((16, 1), u, jnp.int32)
            m = lax.gather(m16, lane, dnums, (1,),
                           mode=lax.GatherScatterMode.PROMISE_IN_BOUNDS)
            ms = m * _SCALE
            for j in range(_EMB // 16):
                sl = pl.ds(j * 16, 16)
                buf[r, sl] = buf[r, sl] * ms + pe_v[r, sl] * m
        return carry

    lax.fori_loop(0, _CHUNK // 16, row_block, 0)


def _body(table, xflat, mflat, pe2, out,
          idx_v, mask_v, pe_v, rows0, rows1, g0, g1, o0, o1):
    nc = 2
    wid = lax.axis_index("s") * nc + lax.axis_index("c")
    base = wid * _PER_W

    pltpu.sync_copy(xflat.at[pl.ds(base, _PER_W)], idx_v)
    pltpu.sync_copy(mflat.at[pl.ds(base, _PER_W)], mask_v)
    pltpu.sync_copy(pe2, pe_v)

    pltpu.async_copy(table.at[idx_v.at[pl.ds(0, _CHUNK)]], rows0, g0)

    def pair(k, carry):
        off0 = 2 * k * _CHUNK
        off1 = off0 + _CHUNK
        off2 = off0 + 2 * _CHUNK

        pltpu.make_async_copy(table.at[pl.ds(0, _CHUNK)], rows0, g0).wait()

        @pl.when(k > 0)
        def _():
            pltpu.make_async_copy(rows1, out.at[pl.ds(0, _CHUNK)], o1).wait()

        pltpu.async_copy(table.at[idx_v.at[pl.ds(off1, _CHUNK)]], rows1, g1)

        _compute_chunk(rows0, off0, mask_v, pe_v)
        pltpu.async_copy(rows0, out.at[pl.ds(base + off0, _CHUNK)], o0)

        pltpu.make_async_copy(table.at[pl.ds(0, _CHUNK)], rows1, g1).wait()
        pltpu.make_async_copy(rows0, out.at[pl.ds(0, _CHUNK)], o0).wait()

        @pl.when(k < _NPAIR - 1)
        def _():
            pltpu.async_copy(table.at[idx_v.at[pl.ds(off2, _CHUNK)]], rows0, g0)

        _compute_chunk(rows1, off1, mask_v, pe_v)
        pltpu.async_copy(rows1, out.at[pl.ds(base + off1, _CHUNK)], o1)
        return carry

    lax.fori_loop(0, _NPAIR, pair, 0)
    pltpu.make_async_copy(rows1, out.at[pl.ds(0, _CHUNK)], o1).wait()


_emb_lookup = pl.kernel(
    _body,
    out_type=jax.ShapeDtypeStruct((_FLAT, _EMB), jnp.float32),
    mesh=plsc.VectorSubcoreMesh(core_axis_name="c", subcore_axis_name="s"),
    scratch_types=[
        pltpu.VMEM((_PER_W,), jnp.int32),
        pltpu.VMEM((_PER_W,), jnp.float32),
        pltpu.VMEM((_CHUNK, _EMB), jnp.float32),
        pltpu.VMEM((_CHUNK, _EMB), jnp.float32),
        pltpu.VMEM((_CHUNK, _EMB), jnp.float32),
        pltpu.SemaphoreType.DMA,
        pltpu.SemaphoreType.DMA,
        pltpu.SemaphoreType.DMA,
        pltpu.SemaphoreType.DMA,
    ],
    compiler_params=pltpu.CompilerParams(use_tc_tiling_on_sc=False),
)


@jax.jit
def kernel(x, attention_mask, table):
    xflat = x.reshape(_FLAT)
    mflat = attention_mask.reshape(_FLAT)
    pe2 = jnp.asarray(_PE2)
    out = _emb_lookup(table, xflat, mflat, pe2)
    return out.reshape(_BATCH, _SEQ, _EMB)
